# baseline (speedup 1.0000x reference)
import functools

import jax
import jax.numpy as jnp
from jax import lax
from jax.experimental import pallas as pl
from jax.experimental.pallas import tpu as pltpu

_B = 32
_S = 8
_N = 80
_D = 256
_V = 1024
_K = 16
_HOPS = 3
_CAP = 64

_NEG = -float("inf")


def _gelu(x):
    return x * 0.5 * (1.0 + lax.erf(x * 0.7071067811865476))


def _sigmoid(x):
    return 1.0 / (1.0 + jnp.exp(-x))


def _dot_t(a, b):
    return lax.dot_general(a, b, (((1,), (1,)), ((), ())),
                           preferred_element_type=jnp.float32)


def _dot_t16(a, b):
    return lax.dot_general(a.astype(jnp.bfloat16), b.astype(jnp.bfloat16),
                           (((1,), (1,)), ((), ())),
                           preferred_element_type=jnp.float32)


def _brain_body(C, E2,
                ids_ref, embed_ref, pos_ref, outW_ref, outb_ref,
                rW_ref, rb_ref, tW_hbm, tb_ref, gW_ref, gb_ref,
                cW_hbm, srcv_ref, dstv_ref,
                out_ref,
                tw_s, state, nxt, cw_buf, cw_ring, einfo_v,
                active_s, einfo_s,
                sem, sem2, csem, sem3):
    t = pl.program_id(0)

    ids_all = ids_ref[...]
    lane_s = lax.broadcasted_iota(jnp.int32, (_B, _S), 1)
    ids_blk = jnp.sum(jnp.where(lane_s == t, ids_all, 0), axis=1,
                      keepdims=True)
    vocab_iota = lax.broadcasted_iota(jnp.int32, (_B, _V), 1)
    onehot = (vocab_iota == ids_blk).astype(jnp.float32)
    x_t = jnp.dot(onehot, embed_ref[...],
                  preferred_element_type=jnp.float32)
    x_t = x_t + pos_ref[pl.ds(t, 1), :]

    scores = _dot_t(x_t, rW_ref[...]) + rb_ref[...]
    lane_n = lax.broadcasted_iota(jnp.int32, (_B, _N), 1)

    rank = jnp.zeros((_B, _N), jnp.int32)
    for m in range(_N):
        col = scores[:, m:m + 1]
        beats = (col > scores) | ((col == scores) & (m < lane_n))
        rank = rank + jnp.where(beats, 1, 0)

    r0 = rank[0:1, :]
    lane0 = lane_n[0:1, :]
    for i in range(_K):
        active_s[i] = jnp.sum(jnp.where(r0 == i, lane0, 0))

    for i in range(_K):
        aid = active_s[i]
        pltpu.make_async_copy(tW_hbm.at[aid], tw_s.at[i], sem.at[i]).start()

    is_top = rank < _K
    m0 = jnp.sum(jnp.where(rank == 0, scores, 0.0), axis=1, keepdims=True)
    ev = jnp.where(is_top, jnp.exp(scores - m0), 0.0)
    evn = ev / jnp.sum(ev, axis=1, keepdims=True)
    wcols = [jnp.sum(jnp.where(rank == i, evn, 0.0), axis=1, keepdims=True)
             for i in range(_K)]
    weights = jnp.concatenate(wcols, axis=1)

    e_iota = lax.broadcasted_iota(jnp.int32, (E2, _N), 1)
    src_oh = e_iota == srcv_ref[...]
    dst_oh = e_iota == dstv_ref[...]
    top_row = jnp.where(r0 < _K, 1, 0)
    src_act = jnp.sum(jnp.where(src_oh, top_row, 0), axis=1, keepdims=True)
    dst_act = jnp.sum(jnp.where(dst_oh, top_row, 0), axis=1, keepdims=True)
    src_slot = jnp.sum(jnp.where(src_oh, r0, 0), axis=1, keepdims=True)
    dst_slot = jnp.sum(jnp.where(dst_oh, r0, 0), axis=1, keepdims=True)
    fired_v = src_act * dst_act
    n_fired = jnp.sum(fired_v)

    row_e = lax.broadcasted_iota(jnp.int32, (E2, E2), 0)
    lane_e = lax.broadcasted_iota(jnp.int32, (E2, E2), 1)
    lmat = jnp.where(lane_e <= row_e, 1.0, 0.0)
    firedrep = fired_v.astype(jnp.float32) * jnp.ones((1, 8), jnp.float32)
    p = lax.dot_general(lmat, firedrep, (((1,), (0,)), ((), ())),
                        preferred_element_type=jnp.float32)[:, 0:1]
    pos = p.astype(jnp.int32) - 1
    pm = jnp.where((fired_v == 1) & (lane_e == pos), 1.0, 0.0)
    eidx = lax.broadcasted_iota(jnp.int32, (E2, 8), 0)
    data = jnp.concatenate(
        [src_slot, dst_slot, eidx[:, 0:6]], axis=1).astype(jnp.float32)
    comp = lax.dot_general(pm, data, (((0,), (0,)), ((), ())),
                           preferred_element_type=jnp.float32)
    einfo_v[...] = comp.astype(jnp.int32)
    cp3 = pltpu.make_async_copy(einfo_v, einfo_s, sem3)
    cp3.start()
    cp3.wait()

    n_res = jnp.minimum(n_fired, _CAP)

    def _cstart(k, _):
        pltpu.make_async_copy(cW_hbm.at[einfo_s[k, 2]], cw_ring.at[k],
                              csem.at[k]).start()
        return 0

    lax.fori_loop(0, n_res, _cstart, 0)

    for i in range(_K):
        aid = active_s[i]
        pltpu.make_async_copy(tW_hbm.at[aid], tw_s.at[i], sem.at[i]).wait()
        pre = _dot_t16(x_t, tw_s[i]) + tb_ref[pl.ds(aid, 1), :]
        gate = _sigmoid(jnp.sum(x_t * gW_ref[pl.ds(aid, 1), :],
                                axis=1, keepdims=True) + gb_ref[aid])
        act = _gelu(pre) * gate * weights[:, i:i + 1]
        state[32 * i:32 * (i + 1), :] = act

    def _edge_compute(k, cw):
        i = einfo_s[k, 0]
        j = einfo_s[k, 1]
        d_id = active_s[j]
        delivered = state[pl.ds(i * 32, 32), :]
        signal = _dot_t16(delivered, cw)
        pre = _dot_t16(signal, tw_s[j]) + tb_ref[pl.ds(d_id, 1), :]
        gate = _sigmoid(jnp.sum(signal * gW_ref[pl.ds(d_id, 1), :],
                                axis=1, keepdims=True) + gb_ref[d_id])
        act = _gelu(pre) * gate
        base = j * 32
        nxt[pl.ds(base, 32), :] = nxt[pl.ds(base, 32), :] + act

    def _res_body_wait(k, _):
        pltpu.make_async_copy(cW_hbm.at[einfo_s[k, 2]], cw_ring.at[k],
                              csem.at[k]).wait()
        _edge_compute(k, cw_ring[k])
        return 0

    def _res_body(k, _):
        _edge_compute(k, cw_ring[k])
        return 0

    def _stream_body(k, _):
        cp = pltpu.make_async_copy(cW_hbm.at[einfo_s[k, 2]], cw_buf, sem2)
        cp.start()
        cp.wait()
        _edge_compute(k, cw_buf[...])
        return 0

    for hop in range(_HOPS - 1):
        nxt[...] = jnp.zeros((_K * _B, _D), jnp.float32)
        lax.fori_loop(0, n_res, _res_body_wait if hop == 0 else _res_body, 0)
        lax.fori_loop(n_res, n_fired, _stream_body, 0)
        state[...] = state[...] + 0.5 * nxt[...]

    acc = state[0:32, :]
    for i in range(1, _K):
        acc = acc + state[32 * i:32 * (i + 1), :]
    mean = acc * (1.0 / _K)
    logits = _dot_t(mean, outW_ref[...]) + outb_ref[...]
    out_ref[...] = logits.reshape(1, _B, _V)


def kernel(input_ids, embed, pos_embed, out_W, out_b, router_W, router_b,
           neu_tW, neu_tb, neu_gW, neu_gb, conn_W, conn_src, conn_dst):
    C = int(conn_src.shape[0])
    E2 = ((C + 7) // 8) * 8
    ids = input_ids.astype(jnp.int32)
    pad = jnp.full((E2 - C,), _N, jnp.int32)
    srcv = jnp.concatenate([conn_src.astype(jnp.int32), pad]).reshape(E2, 1)
    dstv = jnp.concatenate([conn_dst.astype(jnp.int32), pad]).reshape(E2, 1)
    rb2 = router_b.reshape(1, _N)
    gb2 = neu_gb
    outb2 = out_b.reshape(1, _V)

    def full(arr):
        nd = arr.ndim
        return pl.BlockSpec(arr.shape, lambda *_: (0,) * nd)

    out = pl.pallas_call(
        functools.partial(_brain_body, C, E2),
        grid=(_S,),
        in_specs=[
            full(ids),
            full(embed),
            full(pos_embed),
            full(out_W),
            full(outb2),
            full(router_W),
            full(rb2),
            pl.BlockSpec(memory_space=pl.ANY),
            full(neu_tb),
            full(neu_gW),
            pl.BlockSpec(memory_space=pltpu.MemorySpace.SMEM),
            pl.BlockSpec(memory_space=pl.ANY),
            full(srcv),
            full(dstv),
        ],
        out_specs=pl.BlockSpec((1, _B, _V), lambda t: (t, 0, 0)),
        out_shape=jax.ShapeDtypeStruct((_S, _B, _V), jnp.float32),
        scratch_shapes=[
            pltpu.VMEM((_K, _D, _D), jnp.float32),
            pltpu.VMEM((_K * _B, _D), jnp.float32),
            pltpu.VMEM((_K * _B, _D), jnp.float32),
            pltpu.VMEM((_D, _D), jnp.float32),
            pltpu.VMEM((_CAP, _D, _D), jnp.float32),
            pltpu.VMEM((E2, 8), jnp.int32),
            pltpu.SMEM((_K,), jnp.int32),
            pltpu.SMEM((E2, 8), jnp.int32),
            pltpu.SemaphoreType.DMA((_K,)),
            pltpu.SemaphoreType.DMA,
            pltpu.SemaphoreType.DMA((_CAP,)),
            pltpu.SemaphoreType.DMA,
        ],
    )(ids, embed, pos_embed, out_W, outb2, router_W, rb2,
      neu_tW, neu_tb, neu_gW, gb2, conn_W, srcv, dstv)
    return jnp.transpose(out, (1, 0, 2))


# speedup vs baseline: 1.0162x; 1.0162x over previous
import functools

import jax
import jax.numpy as jnp
from jax import lax
from jax.experimental import pallas as pl
from jax.experimental.pallas import tpu as pltpu

_B = 32
_S = 8
_BS = _B * _S
_N = 80
_D = 256
_V = 1024
_K = 16
_HOPS = 3
_CAP = 64


def _gelu(x):
    return x * 0.5 * (1.0 + lax.erf(x * 0.7071067811865476))


def _sigmoid(x):
    return 1.0 / (1.0 + jnp.exp(-x))


def _dot_t(a, b):
    return lax.dot_general(a, b, (((1,), (1,)), ((), ())),
                           preferred_element_type=jnp.float32)


def _dot_t16(a, b):
    return lax.dot_general(a.astype(jnp.bfloat16), b.astype(jnp.bfloat16),
                           (((1,), (1,)), ((), ())),
                           preferred_element_type=jnp.float32)


def _brain_body(C, E2,
                idsv_ref, embed_ref, posrep_ref, outW_ref, outb_ref,
                rW_ref, rb_ref, tW_hbm, tb_ref, gW_ref, gb_ref,
                cW_hbm, srcv_ref, dstv_ref,
                out_ref,
                tw_s, state, nxt, cw_buf, cw_ring,
                x_all_s, w_all_s, r0_all_s, einfo_v,
                active_s, einfo_s,
                sem, sem2, csem, sem3):
    t = pl.program_id(0)

    @pl.when(t == 0)
    def _frontend():
        idsv = idsv_ref[...]
        onehot = (lax.broadcasted_iota(jnp.int32, (_BS, _V), 1)
                  == idsv).astype(jnp.float32)
        x_all = jnp.dot(onehot, embed_ref[...],
                        preferred_element_type=jnp.float32) + posrep_ref[...]
        x_all_s[...] = x_all

        scores = _dot_t(x_all, rW_ref[...]) + rb_ref[...]
        lane_n = lax.broadcasted_iota(jnp.int32, (_BS, _N), 1)

        rank = jnp.zeros((_BS, _N), jnp.int32)
        for m in range(_N):
            col = scores[:, m:m + 1]
            beats = (col > scores) | ((col == scores) & (m < lane_n))
            rank = rank + jnp.where(beats, 1, 0)

        is_top = rank < _K
        m0 = jnp.sum(jnp.where(rank == 0, scores, 0.0), axis=1,
                     keepdims=True)
        ev = jnp.where(is_top, jnp.exp(scores - m0), 0.0)
        evn = ev / jnp.sum(ev, axis=1, keepdims=True)
        wcols = [jnp.sum(jnp.where(rank == i, evn, 0.0), axis=1,
                         keepdims=True) for i in range(_K)]
        w_all_s[...] = jnp.concatenate(wcols, axis=1)

        r0_all = jnp.concatenate(
            [rank[_B * tt:_B * tt + 1, :] for tt in range(_S)], axis=0)
        r0_all_s[...] = r0_all

    r0 = r0_all_s[pl.ds(t, 1), :]
    lane0 = lax.broadcasted_iota(jnp.int32, (1, _N), 1)
    for i in range(_K):
        active_s[i] = jnp.sum(jnp.where(r0 == i, lane0, 0))

    for i in range(_K):
        aid = active_s[i]
        pltpu.make_async_copy(tW_hbm.at[aid], tw_s.at[i], sem.at[i]).start()

    e_iota = lax.broadcasted_iota(jnp.int32, (E2, _N), 1)
    src_oh = e_iota == srcv_ref[...]
    dst_oh = e_iota == dstv_ref[...]
    top_row = jnp.where(r0 < _K, 1, 0)
    src_act = jnp.sum(jnp.where(src_oh, top_row, 0), axis=1, keepdims=True)
    dst_act = jnp.sum(jnp.where(dst_oh, top_row, 0), axis=1, keepdims=True)
    src_slot = jnp.sum(jnp.where(src_oh, r0, 0), axis=1, keepdims=True)
    dst_slot = jnp.sum(jnp.where(dst_oh, r0, 0), axis=1, keepdims=True)
    fired_v = src_act * dst_act
    n_fired = jnp.sum(fired_v)

    row_e = lax.broadcasted_iota(jnp.int32, (E2, E2), 0)
    lane_e = lax.broadcasted_iota(jnp.int32, (E2, E2), 1)
    lmat = jnp.where(lane_e <= row_e, 1.0, 0.0)
    firedrep = fired_v.astype(jnp.float32) * jnp.ones((1, 8), jnp.float32)
    p = lax.dot_general(lmat, firedrep, (((1,), (0,)), ((), ())),
                        preferred_element_type=jnp.float32)[:, 0:1]
    pos = p.astype(jnp.int32) - 1
    pm = jnp.where((fired_v == 1) & (lane_e == pos), 1.0, 0.0)
    eidx = lax.broadcasted_iota(jnp.int32, (E2, 8), 0)
    data = jnp.concatenate(
        [src_slot, dst_slot, eidx[:, 0:6]], axis=1).astype(jnp.float32)
    comp = lax.dot_general(pm, data, (((0,), (0,)), ((), ())),
                           preferred_element_type=jnp.float32)
    einfo_v[...] = comp.astype(jnp.int32)
    cp3 = pltpu.make_async_copy(einfo_v, einfo_s, sem3)
    cp3.start()
    cp3.wait()

    n_res = jnp.minimum(n_fired, _CAP)

    def _cstart(k, _):
        pltpu.make_async_copy(cW_hbm.at[einfo_s[k, 2]], cw_ring.at[k],
                              csem.at[k]).start()
        return 0

    lax.fori_loop(0, n_res, _cstart, 0)

    x_t = x_all_s[pl.ds(t * _B, _B), :]
    weights = w_all_s[pl.ds(t * _B, _B), :]

    for i in range(_K):
        aid = active_s[i]
        pltpu.make_async_copy(tW_hbm.at[aid], tw_s.at[i], sem.at[i]).wait()
        pre = _dot_t16(x_t, tw_s[i]) + tb_ref[pl.ds(aid, 1), :]
        gate = _sigmoid(jnp.sum(x_t * gW_ref[pl.ds(aid, 1), :],
                                axis=1, keepdims=True) + gb_ref[aid])
        act = _gelu(pre) * gate * weights[:, i:i + 1]
        state[32 * i:32 * (i + 1), :] = act

    def _edge_compute(k, cw):
        i = einfo_s[k, 0]
        j = einfo_s[k, 1]
        d_id = active_s[j]
        delivered = state[pl.ds(i * 32, 32), :]
        signal = _dot_t16(delivered, cw)
        pre = _dot_t16(signal, tw_s[j]) + tb_ref[pl.ds(d_id, 1), :]
        gate = _sigmoid(jnp.sum(signal * gW_ref[pl.ds(d_id, 1), :],
                                axis=1, keepdims=True) + gb_ref[d_id])
        act = _gelu(pre) * gate
        base = j * 32
        nxt[pl.ds(base, 32), :] = nxt[pl.ds(base, 32), :] + act
        return 0

    def _res_body_wait(k, _):
        pltpu.make_async_copy(cW_hbm.at[einfo_s[k, 2]], cw_ring.at[k],
                              csem.at[k]).wait()
        return _edge_compute(k, cw_ring[k])

    def _res_body(k, _):
        return _edge_compute(k, cw_ring[k])

    def _stream_body(k, _):
        cp = pltpu.make_async_copy(cW_hbm.at[einfo_s[k, 2]], cw_buf,
                                   sem2)
        cp.start()
        cp.wait()
        return _edge_compute(k, cw_buf[...])

    for hop in range(_HOPS - 1):
        nxt[...] = jnp.zeros((_K * _B, _D), jnp.float32)
        lax.fori_loop(0, n_res, _res_body_wait if hop == 0 else _res_body, 0)
        lax.fori_loop(n_res, n_fired, _stream_body, 0)
        state[...] = state[...] + 0.5 * nxt[...]

    acc = state[0:32, :]
    for i in range(1, _K):
        acc = acc + state[32 * i:32 * (i + 1), :]
    mean = acc * (1.0 / _K)
    logits = _dot_t(mean, outW_ref[...]) + outb_ref[...]
    out_ref[...] = logits.reshape(1, _B, _V)


def kernel(input_ids, embed, pos_embed, out_W, out_b, router_W, router_b,
           neu_tW, neu_tb, neu_gW, neu_gb, conn_W, conn_src, conn_dst):
    C = int(conn_src.shape[0])
    E2 = ((C + 7) // 8) * 8
    idsv = input_ids.astype(jnp.int32).T.reshape(_BS, 1)
    posrep = jnp.repeat(pos_embed, _B, axis=0)
    pad = jnp.full((E2 - C,), _N, jnp.int32)
    srcv = jnp.concatenate([conn_src.astype(jnp.int32), pad]).reshape(E2, 1)
    dstv = jnp.concatenate([conn_dst.astype(jnp.int32), pad]).reshape(E2, 1)
    rb2 = router_b.reshape(1, _N)
    outb2 = out_b.reshape(1, _V)

    def full(arr):
        nd = arr.ndim
        return pl.BlockSpec(arr.shape, lambda *_: (0,) * nd)

    out = pl.pallas_call(
        functools.partial(_brain_body, C, E2),
        grid=(_S,),
        in_specs=[
            full(idsv),
            full(embed),
            full(posrep),
            full(out_W),
            full(outb2),
            full(router_W),
            full(rb2),
            pl.BlockSpec(memory_space=pl.ANY),
            full(neu_tb),
            full(neu_gW),
            pl.BlockSpec(memory_space=pltpu.MemorySpace.SMEM),
            pl.BlockSpec(memory_space=pl.ANY),
            full(srcv),
            full(dstv),
        ],
        out_specs=pl.BlockSpec((1, _B, _V), lambda t: (t, 0, 0)),
        out_shape=jax.ShapeDtypeStruct((_S, _B, _V), jnp.float32),
        scratch_shapes=[
            pltpu.VMEM((_K, _D, _D), jnp.float32),
            pltpu.VMEM((_K * _B, _D), jnp.float32),
            pltpu.VMEM((_K * _B, _D), jnp.float32),
            pltpu.VMEM((_D, _D), jnp.float32),
            pltpu.VMEM((_CAP, _D, _D), jnp.float32),
            pltpu.VMEM((_BS, _D), jnp.float32),
            pltpu.VMEM((_BS, _K), jnp.float32),
            pltpu.VMEM((_S, _N), jnp.int32),
            pltpu.VMEM((E2, 8), jnp.int32),
            pltpu.SMEM((_K,), jnp.int32),
            pltpu.SMEM((E2, 8), jnp.int32),
            pltpu.SemaphoreType.DMA((_K,)),
            pltpu.SemaphoreType.DMA,
            pltpu.SemaphoreType.DMA((_CAP,)),
            pltpu.SemaphoreType.DMA,
        ],
    )(idsv, embed, posrep, out_W, outb2, router_W, rb2,
      neu_tW, neu_tb, neu_gW, neu_gb, conn_W, srcv, dstv)
    return jnp.transpose(out, (1, 0, 2))


# speedup vs baseline: 1.1790x; 1.1602x over previous
import functools

import jax
import jax.numpy as jnp
from jax import lax
from jax.experimental import pallas as pl
from jax.experimental.pallas import tpu as pltpu

_B = 32
_S = 8
_BS = _B * _S
_N = 80
_D = 256
_V = 1024
_K = 16
_HOPS = 3
_CAP = 64


def _gelu(x):
    return x * 0.5 * (1.0 + lax.erf(x * 0.7071067811865476))


def _sigmoid(x):
    return 1.0 / (1.0 + jnp.exp(-x))


def _dot_t(a, b):
    return lax.dot_general(a, b, (((1,), (1,)), ((), ())),
                           preferred_element_type=jnp.float32)


def _dot_t16(a, b):
    return lax.dot_general(a.astype(jnp.bfloat16), b.astype(jnp.bfloat16),
                           (((1,), (1,)), ((), ())),
                           preferred_element_type=jnp.float32)


def _brain_body(C, E2,
                idsv_ref, embed_ref, posrep_ref, outW_ref, outb_ref,
                rW_ref, rb_ref, tW_hbm, tb_ref, gW_ref, gb_ref,
                cW_hbm, srcv_ref, dstv_ref,
                out_ref,
                tw_s, state, nxt, cw_buf, cw_ring,
                x_all_s, w_all_s, r0_all_s, einfo_v,
                active_s, einfo_s,
                sem, sem2, csem, sem3):
    t = pl.program_id(0)

    @pl.when(t == 0)
    def _frontend():
        idsv = idsv_ref[...]
        onehot = (lax.broadcasted_iota(jnp.int32, (_BS, _V), 1)
                  == idsv).astype(jnp.float32)
        x_all = jnp.dot(onehot, embed_ref[...],
                        preferred_element_type=jnp.float32) + posrep_ref[...]
        x_all_s[...] = x_all

        scores = _dot_t(x_all, rW_ref[...]) + rb_ref[...]
        lane_n = lax.broadcasted_iota(jnp.int32, (_BS, _N), 1)

        rank = jnp.zeros((_BS, _N), jnp.int32)
        for m in range(_N):
            col = scores[:, m:m + 1]
            beats = (col > scores) | ((col == scores) & (m < lane_n))
            rank = rank + jnp.where(beats, 1, 0)

        is_top = rank < _K
        m0 = jnp.sum(jnp.where(rank == 0, scores, 0.0), axis=1,
                     keepdims=True)
        ev = jnp.where(is_top, jnp.exp(scores - m0), 0.0)
        evn = ev / jnp.sum(ev, axis=1, keepdims=True)
        wcols = [jnp.sum(jnp.where(rank == i, evn, 0.0), axis=1,
                         keepdims=True) for i in range(_K)]
        w_all_s[...] = jnp.concatenate(wcols, axis=1)

        r0_all = jnp.concatenate(
            [rank[_B * tt:_B * tt + 1, :] for tt in range(_S)], axis=0)
        r0_all_s[...] = r0_all

    r0 = r0_all_s[pl.ds(t, 1), :]
    lane0 = lax.broadcasted_iota(jnp.int32, (1, _N), 1)
    for i in range(_K):
        active_s[i] = jnp.sum(jnp.where(r0 == i, lane0, 0))

    for i in range(_K):
        aid = active_s[i]
        pltpu.make_async_copy(tW_hbm.at[aid], tw_s.at[i], sem.at[i]).start()

    e_iota = lax.broadcasted_iota(jnp.int32, (E2, _N), 1)
    src_oh = e_iota == srcv_ref[...]
    dst_oh = e_iota == dstv_ref[...]
    top_row = jnp.where(r0 < _K, 1, 0)
    src_act = jnp.sum(jnp.where(src_oh, top_row, 0), axis=1, keepdims=True)
    dst_act = jnp.sum(jnp.where(dst_oh, top_row, 0), axis=1, keepdims=True)
    src_slot = jnp.sum(jnp.where(src_oh, r0, 0), axis=1, keepdims=True)
    dst_slot = jnp.sum(jnp.where(dst_oh, r0, 0), axis=1, keepdims=True)
    fired_v = src_act * dst_act
    n_fired = jnp.sum(fired_v)

    row_e = lax.broadcasted_iota(jnp.int32, (E2, E2), 0)
    lane_e = lax.broadcasted_iota(jnp.int32, (E2, E2), 1)
    lmat = jnp.where(lane_e <= row_e, 1.0, 0.0)
    firedrep = fired_v.astype(jnp.float32) * jnp.ones((1, 8), jnp.float32)
    p = lax.dot_general(lmat, firedrep, (((1,), (0,)), ((), ())),
                        preferred_element_type=jnp.float32)[:, 0:1]
    pos = p.astype(jnp.int32) - 1
    pm = jnp.where((fired_v == 1) & (lane_e == pos), 1.0, 0.0)
    eidx = lax.broadcasted_iota(jnp.int32, (E2, 8), 0)
    data = jnp.concatenate(
        [src_slot, dst_slot, eidx[:, 0:6]], axis=1).astype(jnp.float32)
    comp = lax.dot_general(pm, data, (((0,), (0,)), ((), ())),
                           preferred_element_type=jnp.float32)
    einfo_v[...] = comp.astype(jnp.int32)
    cp3 = pltpu.make_async_copy(einfo_v, einfo_s, sem3)
    cp3.start()
    cp3.wait()

    n_res = jnp.minimum(n_fired, _CAP)
    n_resp = jnp.minimum((n_res + 1) // 2 * 2, _CAP)

    def _cstart(k, _):
        pltpu.make_async_copy(cW_hbm.at[einfo_s[k, 2]], cw_ring.at[k],
                              csem.at[k]).start()
        return 0

    lax.fori_loop(0, n_resp, _cstart, 0)

    x_t = x_all_s[pl.ds(t * _B, _B), :]
    weights = w_all_s[pl.ds(t * _B, _B), :]

    for i in range(_K):
        aid = active_s[i]
        pltpu.make_async_copy(tW_hbm.at[aid], tw_s.at[i], sem.at[i]).wait()
        pre = _dot_t16(x_t, tw_s[i]) + tb_ref[pl.ds(aid, 1), :]
        gate = _sigmoid(jnp.sum(x_t * gW_ref[pl.ds(aid, 1), :],
                                axis=1, keepdims=True) + gb_ref[aid])
        act = _gelu(pre) * gate * weights[:, i:i + 1]
        state[32 * i:32 * (i + 1), :] = act

    def _edge_act(k, cw, scale):
        i = einfo_s[k, 0]
        j = einfo_s[k, 1]
        d_id = active_s[j]
        delivered = state[pl.ds(i * 32, 32), :]
        signal = _dot_t16(delivered, cw)
        pre = _dot_t16(signal, tw_s[j]) + tb_ref[pl.ds(d_id, 1), :]
        gate = _sigmoid(jnp.sum(signal * gW_ref[pl.ds(d_id, 1), :],
                                axis=1, keepdims=True) + gb_ref[d_id])
        return _gelu(pre) * gate * scale, j

    def _accum(act, j):
        base = j * 32
        nxt[pl.ds(base, 32), :] = nxt[pl.ds(base, 32), :] + act

    def _pair_body(wait_first):
        def body(kk, _):
            k1 = 2 * kk
            k2 = k1 + 1
            if wait_first:
                pltpu.make_async_copy(cW_hbm.at[einfo_s[k1, 2]],
                                      cw_ring.at[k1], csem.at[k1]).wait()
                pltpu.make_async_copy(cW_hbm.at[einfo_s[k2, 2]],
                                      cw_ring.at[k2], csem.at[k2]).wait()
            m1 = jnp.where(k1 < n_res, 1.0, 0.0)
            m2 = jnp.where(k2 < n_res, 1.0, 0.0)
            act1, j1 = _edge_act(k1, cw_ring[k1], m1)
            act2, j2 = _edge_act(k2, cw_ring[k2], m2)
            _accum(act1, j1)
            _accum(act2, j2)
            return 0
        return body

    def _stream_body(k, _):
        cp = pltpu.make_async_copy(cW_hbm.at[einfo_s[k, 2]], cw_buf,
                                   sem2)
        cp.start()
        cp.wait()
        act, j = _edge_act(k, cw_buf[...], 1.0)
        _accum(act, j)
        return 0

    for hop in range(_HOPS - 1):
        nxt[...] = jnp.zeros((_K * _B, _D), jnp.float32)
        lax.fori_loop(0, n_resp // 2, _pair_body(hop == 0), 0)
        lax.fori_loop(n_res, n_fired, _stream_body, 0)
        state[...] = state[...] + 0.5 * nxt[...]

    acc = state[0:32, :]
    for i in range(1, _K):
        acc = acc + state[32 * i:32 * (i + 1), :]
    mean = acc * (1.0 / _K)
    logits = _dot_t(mean, outW_ref[...]) + outb_ref[...]
    out_ref[...] = logits.reshape(1, _B, _V)


def kernel(input_ids, embed, pos_embed, out_W, out_b, router_W, router_b,
           neu_tW, neu_tb, neu_gW, neu_gb, conn_W, conn_src, conn_dst):
    C = int(conn_src.shape[0])
    E2 = ((C + 7) // 8) * 8
    idsv = input_ids.astype(jnp.int32).T.reshape(_BS, 1)
    posrep = jnp.repeat(pos_embed, _B, axis=0)
    pad = jnp.full((E2 - C,), _N, jnp.int32)
    srcv = jnp.concatenate([conn_src.astype(jnp.int32), pad]).reshape(E2, 1)
    dstv = jnp.concatenate([conn_dst.astype(jnp.int32), pad]).reshape(E2, 1)
    rb2 = router_b.reshape(1, _N)
    outb2 = out_b.reshape(1, _V)

    def full(arr):
        nd = arr.ndim
        return pl.BlockSpec(arr.shape, lambda *_: (0,) * nd)

    out = pl.pallas_call(
        functools.partial(_brain_body, C, E2),
        grid=(_S,),
        in_specs=[
            full(idsv),
            full(embed),
            full(posrep),
            full(out_W),
            full(outb2),
            full(router_W),
            full(rb2),
            pl.BlockSpec(memory_space=pl.ANY),
            full(neu_tb),
            full(neu_gW),
            pl.BlockSpec(memory_space=pltpu.MemorySpace.SMEM),
            pl.BlockSpec(memory_space=pl.ANY),
            full(srcv),
            full(dstv),
        ],
        out_specs=pl.BlockSpec((1, _B, _V), lambda t: (t, 0, 0)),
        out_shape=jax.ShapeDtypeStruct((_S, _B, _V), jnp.float32),
        scratch_shapes=[
            pltpu.VMEM((_K, _D, _D), jnp.float32),
            pltpu.VMEM((_K * _B, _D), jnp.float32),
            pltpu.VMEM((_K * _B, _D), jnp.float32),
            pltpu.VMEM((_D, _D), jnp.float32),
            pltpu.VMEM((_CAP, _D, _D), jnp.float32),
            pltpu.VMEM((_BS, _D), jnp.float32),
            pltpu.VMEM((_BS, _K), jnp.float32),
            pltpu.VMEM((_S, _N), jnp.int32),
            pltpu.VMEM((E2, 8), jnp.int32),
            pltpu.SMEM((_K,), jnp.int32),
            pltpu.SMEM((E2, 8), jnp.int32),
            pltpu.SemaphoreType.DMA((_K,)),
            pltpu.SemaphoreType.DMA,
            pltpu.SemaphoreType.DMA((_CAP,)),
            pltpu.SemaphoreType.DMA,
        ],
    )(idsv, embed, posrep, out_W, outb2, router_W, rb2,
      neu_tW, neu_tb, neu_gW, neu_gb, conn_W, srcv, dstv)
    return jnp.transpose(out, (1, 0, 2))
